# Initial kernel scaffold; baseline (speedup 1.0000x reference)
import jax
import jax.numpy as jnp
from jax.experimental import pallas as pl


def kernel(leaf_content_id, content_mask, composition_info, emb_table, W, b):
    raise NotImplementedError("write your pallas kernel here")


# speedup vs baseline: 5.3249x; 5.3249x over previous
import functools
import math

import jax
import jax.numpy as jnp
import numpy as np
from jax import lax
from jax.experimental import pallas as pl
from jax.experimental.pallas import tpu as pltpu
from jax.experimental.pallas import tpu_sc as plsc

B = 4096
L = 20
STEPS = L - 1
NODES = 2 * L - 1
D = 64
NCAT = 128
NF = D // 2 + 1

_j = np.arange(D)[:, None].astype(np.float64)
_f = np.arange(NF)[None, :].astype(np.float64)
_ang = 2.0 * math.pi * _j * _f / D
_FR = np.cos(_ang).astype(np.float32)
_FI = (-np.sin(_ang)).astype(np.float32)
_w = np.ones((NF, 1))
_w[1:NF - 1] = 2.0
_GR = (_w * np.cos(_ang.T) / D).astype(np.float32)
_GI = (-_w * np.sin(_ang.T) / D).astype(np.float32)


def _sc_gather(table, ids):
    nids = ids.shape[0]
    info = plsc.get_sparse_core_info()
    nc, ns = info.num_cores, info.num_subcores
    nw = nc * ns
    chunk = 128
    per_w = nids // nw
    nchunks = per_w // chunk
    assert per_w * nw == nids and nchunks * chunk == per_w

    mesh = plsc.VectorSubcoreMesh(core_axis_name="c", subcore_axis_name="s")

    @functools.partial(
        pl.kernel,
        mesh=mesh,
        out_type=jax.ShapeDtypeStruct((nids, D), jnp.float32),
        scratch_types=[
            pltpu.VMEM((chunk,), jnp.int32),
            pltpu.VMEM((chunk, D), jnp.float32),
            pltpu.SemaphoreType.DMA,
        ],
        compiler_params=pltpu.CompilerParams(use_tc_tiling_on_sc=False),
    )
    def gather_k(table_hbm, idx_hbm, out_hbm, idx_v, rows_v, sem):
        wid = lax.axis_index("s") * nc + lax.axis_index("c")
        base = wid * per_w
        for ci in range(nchunks):
            off = base + ci * chunk
            pltpu.sync_copy(idx_hbm.at[pl.ds(off, chunk)], idx_v)
            pltpu.async_copy(table_hbm.at[idx_v], rows_v, sem).wait()
            pltpu.sync_copy(rows_v, out_hbm.at[pl.ds(off, chunk)])

    return gather_k(table, ids)


_BS = 256


def _dot(x, y):
    return jax.lax.dot_general(
        x, y, (((1,), (0,)), ((), ())),
        precision=jax.lax.Precision.HIGHEST,
        preferred_element_type=jnp.float32)


def _tc_body(leaf_ref, mask_ref, loh_ref, roh_ref, poh_ref,
             fr_ref, fi_ref, gr_ref, gi_ref, wt_ref, b_ref, out_ref, v_ref):
    s = pl.program_id(1)

    @pl.when(s == 0)
    def _init():
        leaf = leaf_ref[...] * mask_ref[...][:, :, None]
        nrm = jnp.sqrt(jnp.sum(leaf * leaf, axis=2, keepdims=True)) + 1e-6
        v_ref[:, :L, :] = leaf / nrm
        v_ref[:, L:, :] = jnp.zeros((_BS, NODES - L, D), jnp.float32)

    v = v_ref[...]
    loh = loh_ref[0]
    roh = roh_ref[0]
    poh = poh_ref[0]
    a = jnp.sum(v * loh[:, :, None], axis=1)
    bb = jnp.sum(v * roh[:, :, None], axis=1)
    ar = _dot(a, fr_ref[...])
    ai = _dot(a, fi_ref[...])
    br = _dot(bb, fr_ref[...])
    bi = _dot(bb, fi_ref[...])
    re = ar * br + ai * bi
    im = ar * bi - ai * br
    c = _dot(re, gr_ref[...]) + _dot(im, gi_ref[...])
    c = c / (jnp.sqrt(jnp.sum(c * c, axis=1, keepdims=True)) + 1e-6)
    p3 = poh[:, :, None]
    v_ref[...] = v * (1.0 - p3) + c[:, None, :] * p3

    @pl.when(s == STEPS - 1)
    def _fin():
        logits = jax.lax.dot_general(
            v_ref[...], wt_ref[...], (((2,), (0,)), ((), ())),
            precision=jax.lax.Precision.HIGHEST,
            preferred_element_type=jnp.float32) + b_ref[...][None]
        out_ref[...] = jax.nn.sigmoid(logits)


def _tc_compose(leaf_vec, mask_f, loh, roh, poh, wt, b2):
    grid = (B // _BS, STEPS)
    const = lambda shape: pl.BlockSpec(shape, lambda i, s: (0,) * len(shape))
    oh_spec = pl.BlockSpec((1, _BS, NODES), lambda i, s: (s, i, 0))
    return pl.pallas_call(
        _tc_body,
        grid=grid,
        in_specs=[
            pl.BlockSpec((_BS, L, D), lambda i, s: (i, 0, 0)),
            pl.BlockSpec((_BS, L), lambda i, s: (i, 0)),
            oh_spec,
            oh_spec,
            oh_spec,
            const((D, NF)),
            const((D, NF)),
            const((NF, D)),
            const((NF, D)),
            const((D, NCAT)),
            const((1, NCAT)),
        ],
        out_specs=pl.BlockSpec((_BS, NODES, NCAT), lambda i, s: (i, 0, 0)),
        out_shape=jax.ShapeDtypeStruct((B, NODES, NCAT), jnp.float32),
        scratch_shapes=[pltpu.VMEM((_BS, NODES, D), jnp.float32)],
        compiler_params=pltpu.CompilerParams(
            dimension_semantics=("parallel", "arbitrary")),
    )(leaf_vec, mask_f, loh, roh, poh,
      jnp.asarray(_FR), jnp.asarray(_FI), jnp.asarray(_GR), jnp.asarray(_GI),
      wt, b2)


def kernel(leaf_content_id, content_mask, composition_info, emb_table, W, b):
    ids = leaf_content_id.astype(jnp.int32).reshape(-1)
    leaf_rows = _sc_gather(emb_table, ids)
    leaf_vec = leaf_rows.reshape(B, L, D)
    mask_f = content_mask.astype(jnp.float32)
    ci = composition_info.astype(jnp.int32)
    nids = jnp.arange(NODES, dtype=jnp.int32)[None, None, :]
    cit = jnp.transpose(ci, (1, 0, 2))
    loh = (cit[:, :, 0][:, :, None] == nids).astype(jnp.float32)
    roh = (cit[:, :, 1][:, :, None] == nids).astype(jnp.float32)
    poh = (cit[:, :, 2][:, :, None] == nids).astype(jnp.float32)
    wt = W.astype(jnp.float32).T
    b2 = b.astype(jnp.float32).reshape(1, NCAT)
    return _tc_compose(leaf_vec, mask_f, loh, roh, poh, wt, b2)


# speedup vs baseline: 5.4878x; 1.0306x over previous
import functools
import math

import jax
import jax.numpy as jnp
import numpy as np
from jax import lax
from jax.experimental import pallas as pl
from jax.experimental.pallas import tpu as pltpu
from jax.experimental.pallas import tpu_sc as plsc

B = 4096
L = 20
STEPS = L - 1
NODES = 2 * L - 1
D = 64
NCAT = 128
NF = D // 2 + 1

_j = np.arange(D)[:, None].astype(np.float64)
_f = np.arange(NF)[None, :].astype(np.float64)
_ang = 2.0 * math.pi * _j * _f / D
_FR = np.cos(_ang).astype(np.float32)
_FI = (-np.sin(_ang)).astype(np.float32)
_w = np.ones((NF, 1))
_w[1:NF - 1] = 2.0
_GR = (_w * np.cos(_ang.T) / D).astype(np.float32)
_GI = (-_w * np.sin(_ang.T) / D).astype(np.float32)

_FFA = np.zeros((128, 128), np.float32)
_FFA[0:64, 0:NF] = _FR
_FFA[0:64, 64:64 + NF] = _FI
_FFB = np.zeros((128, 128), np.float32)
_FFB[64:128, 0:NF] = _FR
_FFB[64:128, 64:64 + NF] = _FI
_FFB2 = np.zeros((128, 128), np.float32)
_FFB2[64:128, 0:NF] = _FI
_FFB2[64:128, 64:64 + NF] = _FR
_G1 = np.zeros((128, 128), np.float32)
_G1[0:NF, 0:64] = _GR
_G1[0:NF, 64:128] = _GR
_G1[64:64 + NF, 0:64] = _GR
_G1[64:64 + NF, 64:128] = _GR
_G2 = np.zeros((128, 128), np.float32)
_G2[0:NF, 0:64] = _GI
_G2[0:NF, 64:128] = _GI
_G2[64:64 + NF, 0:64] = -_GI
_G2[64:64 + NF, 64:128] = -_GI
_HALF = np.full((128, 128), 0.5, np.float32)


def _sc_gather(table, ids):
    nids = ids.shape[0]
    info = plsc.get_sparse_core_info()
    nc, ns = info.num_cores, info.num_subcores
    nw = nc * ns
    chunk = 128
    per_w = nids // nw
    nchunks = per_w // chunk
    assert per_w * nw == nids and nchunks * chunk == per_w

    mesh = plsc.VectorSubcoreMesh(core_axis_name="c", subcore_axis_name="s")

    @functools.partial(
        pl.kernel,
        mesh=mesh,
        out_type=jax.ShapeDtypeStruct((nids, D), jnp.float32),
        scratch_types=[
            pltpu.VMEM((chunk,), jnp.int32),
            pltpu.VMEM((chunk, D), jnp.float32),
            pltpu.SemaphoreType.DMA,
        ],
        compiler_params=pltpu.CompilerParams(use_tc_tiling_on_sc=False),
    )
    def gather_k(table_hbm, idx_hbm, out_hbm, idx_v, rows_v, sem):
        wid = lax.axis_index("s") * nc + lax.axis_index("c")
        base = wid * per_w
        for ci in range(nchunks):
            off = base + ci * chunk
            pltpu.sync_copy(idx_hbm.at[pl.ds(off, chunk)], idx_v)
            pltpu.async_copy(table_hbm.at[idx_v], rows_v, sem).wait()
            pltpu.sync_copy(rows_v, out_hbm.at[pl.ds(off, chunk)])

    return gather_k(table, ids)


_BS = 256


def _mm(x, y):
    return jax.lax.dot_general(
        x, y, (((x.ndim - 1,), (0,)), ((), ())),
        precision=jax.lax.Precision.HIGHEST,
        preferred_element_type=jnp.float32)


def _tc_body(leaf_ref, li_ref, ri_ref, pi_ref,
             ffa_ref, ffb_ref, ffb2_ref, g1_ref, g2_ref, half_ref,
             w2_ref, b_ref, out_ref, v_ref):
    s = pl.program_id(1)

    @pl.when(s == 0)
    def _init():
        ld = leaf_ref[...]
        n2 = _mm(ld * ld, half_ref[...])
        v_ref[:, :L, :] = ld / (jnp.sqrt(n2) + 1e-6)
        v_ref[:, L:, :] = jnp.zeros((_BS, NODES - L, 128), jnp.float32)

    vd = v_ref[...]
    li = li_ref[0]
    ri = ri_ref[0]
    pi = pi_ref[0]
    iota1 = lax.broadcasted_iota(jnp.int32, (_BS, NODES, 1), 1)
    lane = lax.broadcasted_iota(jnp.int32, (_BS, NODES, 128), 2)
    lsel = (iota1 == li[:, None, :]).astype(jnp.float32)
    rsel = (iota1 == ri[:, None, :]).astype(jnp.float32)
    lrh = jnp.where(lane < 64, lsel, rsel)
    g = jnp.sum(vd * lrh, axis=1)
    af = _mm(g, ffa_ref[...])
    bf = _mm(g, ffb_ref[...])
    bfr = _mm(g, ffb2_ref[...])
    prod1 = af * bf
    prod2 = af * bfr
    cd = _mm(prod1, g1_ref[...]) + _mm(prod2, g2_ref[...])
    n2 = _mm(cd * cd, half_ref[...])
    cn = cd / (jnp.sqrt(n2) + 1e-6)
    psel = iota1 == pi[:, None, :]
    v_ref[...] = jnp.where(psel, cn[:, None, :], vd)

    @pl.when(s == STEPS - 1)
    def _fin():
        logits = _mm(v_ref[...], w2_ref[...]) + b_ref[...][None]
        out_ref[...] = jax.nn.sigmoid(logits)


def _tc_compose(leaf_dup, li, ri, pi, w2, b2):
    grid = (B // _BS, STEPS)
    const = lambda shape: pl.BlockSpec(shape, lambda i, s: (0,) * len(shape))
    ix_spec = pl.BlockSpec((1, _BS, 1), lambda i, s: (s, i, 0))
    return pl.pallas_call(
        _tc_body,
        grid=grid,
        in_specs=[
            pl.BlockSpec((_BS, L, 128), lambda i, s: (i, 0, 0)),
            ix_spec,
            ix_spec,
            ix_spec,
            const((128, 128)),
            const((128, 128)),
            const((128, 128)),
            const((128, 128)),
            const((128, 128)),
            const((128, 128)),
            const((128, NCAT)),
            const((1, NCAT)),
        ],
        out_specs=pl.BlockSpec((_BS, NODES, NCAT), lambda i, s: (i, 0, 0)),
        out_shape=jax.ShapeDtypeStruct((B, NODES, NCAT), jnp.float32),
        scratch_shapes=[pltpu.VMEM((_BS, NODES, 128), jnp.float32)],
        compiler_params=pltpu.CompilerParams(
            dimension_semantics=("parallel", "arbitrary"),
            vmem_limit_bytes=100 * 1024 * 1024),
    )(leaf_dup, li, ri, pi,
      jnp.asarray(_FFA), jnp.asarray(_FFB), jnp.asarray(_FFB2),
      jnp.asarray(_G1), jnp.asarray(_G2), jnp.asarray(_HALF),
      w2, b2)


def kernel(leaf_content_id, content_mask, composition_info, emb_table, W, b):
    ids = leaf_content_id.astype(jnp.int32).reshape(-1)
    leaf_rows = _sc_gather(emb_table, ids)
    lv = leaf_rows.reshape(B, L, D)
    lv = lv * content_mask.astype(jnp.float32)[:, :, None]
    leaf_dup = jnp.concatenate([lv, lv], axis=2)
    ci = composition_info.astype(jnp.int32)
    cit = jnp.transpose(ci, (1, 0, 2))
    li = cit[:, :, 0][:, :, None]
    ri = cit[:, :, 1][:, :, None]
    pi = cit[:, :, 2][:, :, None]
    wt = W.astype(jnp.float32).T
    w2 = jnp.concatenate([wt, jnp.zeros((64, NCAT), jnp.float32)], axis=0)
    b2 = b.astype(jnp.float32).reshape(1, NCAT)
    return _tc_compose(leaf_dup, li, ri, pi, w2, b2)


# speedup vs baseline: 6.2576x; 1.1403x over previous
import functools
import math

import jax
import jax.numpy as jnp
import numpy as np
from jax import lax
from jax.experimental import pallas as pl
from jax.experimental.pallas import tpu as pltpu
from jax.experimental.pallas import tpu_sc as plsc

B = 4096
L = 20
STEPS = L - 1
NODES = 2 * L - 1
D = 64
NCAT = 128
NF = D // 2 + 1

_j = np.arange(D)[:, None].astype(np.float64)
_f = np.arange(NF)[None, :].astype(np.float64)
_ang = 2.0 * math.pi * _j * _f / D
_FR = np.cos(_ang).astype(np.float32)
_FI = (-np.sin(_ang)).astype(np.float32)
_w = np.ones((NF, 1))
_w[1:NF - 1] = 2.0
_GR = (_w * np.cos(_ang.T) / D).astype(np.float32)
_GI = (-_w * np.sin(_ang.T) / D).astype(np.float32)

_FFA = np.zeros((128, 128), np.float32)
_FFA[0:64, 0:NF] = _FR
_FFA[0:64, 64:64 + NF] = _FI
_FFB = np.zeros((128, 128), np.float32)
_FFB[64:128, 0:NF] = _FR
_FFB[64:128, 64:64 + NF] = _FI
_FFB2 = np.zeros((128, 128), np.float32)
_FFB2[64:128, 0:NF] = _FI
_FFB2[64:128, 64:64 + NF] = _FR
_G1 = np.zeros((128, 128), np.float32)
_G1[0:NF, 0:64] = _GR
_G1[0:NF, 64:128] = _GR
_G1[64:64 + NF, 0:64] = _GR
_G1[64:64 + NF, 64:128] = _GR
_G2 = np.zeros((128, 128), np.float32)
_G2[0:NF, 0:64] = _GI
_G2[0:NF, 64:128] = _GI
_G2[64:64 + NF, 0:64] = -_GI
_G2[64:64 + NF, 64:128] = -_GI
_HALF = np.full((128, 128), 0.5, np.float32)


def _sc_gather(table, ids):
    nids = ids.shape[0]
    info = plsc.get_sparse_core_info()
    nc, ns = info.num_cores, info.num_subcores
    nw = nc * ns
    chunk = 128
    per_w = nids // nw
    nchunks = per_w // chunk
    assert per_w * nw == nids and nchunks * chunk == per_w

    mesh = plsc.VectorSubcoreMesh(core_axis_name="c", subcore_axis_name="s")

    @functools.partial(
        pl.kernel,
        mesh=mesh,
        out_type=jax.ShapeDtypeStruct((nids, D), jnp.float32),
        scratch_types=[
            pltpu.VMEM((chunk,), jnp.int32),
            pltpu.VMEM((chunk, D), jnp.float32),
            pltpu.SemaphoreType.DMA,
        ],
        compiler_params=pltpu.CompilerParams(use_tc_tiling_on_sc=False),
    )
    def gather_k(table_hbm, idx_hbm, out_hbm, idx_v, rows_v, sem):
        wid = lax.axis_index("s") * nc + lax.axis_index("c")
        base = wid * per_w
        for ci in range(nchunks):
            off = base + ci * chunk
            pltpu.sync_copy(idx_hbm.at[pl.ds(off, chunk)], idx_v)
            pltpu.async_copy(table_hbm.at[idx_v], rows_v, sem).wait()
            pltpu.sync_copy(rows_v, out_hbm.at[pl.ds(off, chunk)])

    return gather_k(table, ids)


_BS = 256


def _mm(x, y):
    return jax.lax.dot_general(
        x, y, (((x.ndim - 1,), (0,)), ((), ())),
        precision=jax.lax.Precision.HIGHEST,
        preferred_element_type=jnp.float32)


def _tc_body(leaf_ref, li_ref, ri_ref, pi_ref,
             ffa_ref, ffb_ref, ffb2_ref, g1_ref, g2_ref, half_ref,
             w2_ref, b_ref, out_ref, v_ref):
    s = pl.program_id(1)

    @pl.when(s == 0)
    def _init():
        ld = leaf_ref[...]
        n2 = _mm(ld * ld, half_ref[...])
        v_ref[:L] = ld / (jnp.sqrt(n2) + 1e-6)
        v_ref[L:] = jnp.zeros((NODES - L, _BS, 128), jnp.float32)

    li = li_ref[0]
    ri = ri_ref[0]
    pi = pi_ref[0]
    lane = lax.broadcasted_iota(jnp.int32, (_BS, 128), 1)
    lf = (lane == li).astype(jnp.float32)
    rf = (lane == ri).astype(jnp.float32)
    lo64 = lane < 64

    vd = [v_ref[n] for n in range(NODES)]
    g = jnp.zeros((_BS, 128), jnp.float32)
    for n in range(NODES):
        m = jnp.where(lo64, lf[:, n:n + 1], rf[:, n:n + 1])
        g = g + vd[n] * m

    af = _mm(g, ffa_ref[...])
    bf = _mm(g, ffb_ref[...])
    bfr = _mm(g, ffb2_ref[...])
    prod1 = af * bf
    prod2 = af * bfr
    cd = _mm(prod1, g1_ref[...]) + _mm(prod2, g2_ref[...])
    n2 = _mm(cd * cd, half_ref[...])
    cn = cd / (jnp.sqrt(n2) + 1e-6)

    pb = lane == pi
    for n in range(NODES):
        v_ref[n] = jnp.where(pb[:, n:n + 1], cn, vd[n])

    @pl.when(s == STEPS - 1)
    def _fin():
        sg = jax.nn.sigmoid(_mm(v_ref[...], w2_ref[...]) + b_ref[...][None])
        for n in range(NODES):
            out_ref[:, n, :] = sg[n]


def _tc_compose(leaf_dup, li, ri, pi, w2, b2):
    grid = (B // _BS, STEPS)
    const = lambda shape: pl.BlockSpec(shape, lambda i, s: (0,) * len(shape))
    ix_spec = pl.BlockSpec((1, _BS, 1), lambda i, s: (s, i, 0))
    return pl.pallas_call(
        _tc_body,
        grid=grid,
        in_specs=[
            pl.BlockSpec((L, _BS, 128), lambda i, s: (0, i, 0)),
            ix_spec,
            ix_spec,
            ix_spec,
            const((128, 128)),
            const((128, 128)),
            const((128, 128)),
            const((128, 128)),
            const((128, 128)),
            const((128, 128)),
            const((128, NCAT)),
            const((1, NCAT)),
        ],
        out_specs=pl.BlockSpec((_BS, NODES, NCAT), lambda i, s: (i, 0, 0)),
        out_shape=jax.ShapeDtypeStruct((B, NODES, NCAT), jnp.float32),
        scratch_shapes=[pltpu.VMEM((NODES, _BS, 128), jnp.float32)],
        compiler_params=pltpu.CompilerParams(
            dimension_semantics=("parallel", "arbitrary"),
            vmem_limit_bytes=100 * 1024 * 1024),
    )(leaf_dup, li, ri, pi,
      jnp.asarray(_FFA), jnp.asarray(_FFB), jnp.asarray(_FFB2),
      jnp.asarray(_G1), jnp.asarray(_G2), jnp.asarray(_HALF),
      w2, b2)


def kernel(leaf_content_id, content_mask, composition_info, emb_table, W, b):
    ids = leaf_content_id.astype(jnp.int32).reshape(-1)
    leaf_rows = _sc_gather(emb_table, ids)
    lv = leaf_rows.reshape(B, L, D)
    lv = lv * content_mask.astype(jnp.float32)[:, :, None]
    lt = jnp.transpose(lv, (1, 0, 2))
    leaf_dup = jnp.concatenate([lt, lt], axis=2)
    ci = composition_info.astype(jnp.int32)
    cit = jnp.transpose(ci, (1, 0, 2))
    li = cit[:, :, 0][:, :, None]
    ri = cit[:, :, 1][:, :, None]
    pi = cit[:, :, 2][:, :, None]
    wt = W.astype(jnp.float32).T
    w2 = jnp.concatenate([wt, jnp.zeros((64, NCAT), jnp.float32)], axis=0)
    b2 = b.astype(jnp.float32).reshape(1, NCAT)
    return _tc_compose(leaf_dup, li, ri, pi, w2, b2)
